# baseline (speedup 1.0000x reference)
import functools

import jax
import jax.numpy as jnp
from jax import lax
from jax.experimental import pallas as pl
from jax.experimental.pallas import tpu as pltpu
from jax.experimental.pallas import tpu_sc as plsc

B, L, D = 16, 512, 128
H = B // 2
CHUNK = 128
QPI = L // CHUNK


def _sc_gather_half(table, t1c3, t2c3, b0):
    info = plsc.get_sparse_core_info()
    nc, ns = info.num_cores, info.num_subcores
    mesh = plsc.VectorSubcoreMesh(core_axis_name="c", subcore_axis_name="s")

    @functools.partial(
        pl.kernel,
        mesh=mesh,
        out_type=jax.ShapeDtypeStruct((2, H, L, D), jnp.float32),
        scratch_types=[
            pltpu.VMEM((2, CHUNK), jnp.int32),
            pltpu.VMEM((2 * CHUNK, D), jnp.float32),
            pltpu.SemaphoreType.DMA,
        ],
    )
    def k(table_hbm, t1_hbm, t2_hbm, out_hbm, idx_v, rows_v, sem):
        wid = lax.axis_index("s") * nc + lax.axis_index("c")
        item = wid // QPI
        q = wid % QPI
        pltpu.sync_copy(t1_hbm.at[b0 + item, q], idx_v.at[0])
        pltpu.sync_copy(t2_hbm.at[b0 + item, q], idx_v.at[1])
        c1 = pltpu.async_copy(
            table_hbm.at[idx_v.at[0]], rows_v.at[pl.ds(0, CHUNK)], sem)
        c2 = pltpu.async_copy(
            table_hbm.at[idx_v.at[1]], rows_v.at[pl.ds(CHUNK, CHUNK)], sem)
        c1.wait()
        c2.wait()
        pltpu.sync_copy(rows_v.at[pl.ds(0, CHUNK)],
                        out_hbm.at[0, item, pl.ds(q * CHUNK, CHUNK)])
        pltpu.sync_copy(rows_v.at[pl.ds(CHUNK, CHUNK)],
                        out_hbm.at[1, item, pl.ds(q * CHUNK, CHUNK)])

    return k(table, t1c3, t2c3)


def _tc_body(e1_ref, e2_ref, att_ref, w_ref, b_ref, logit_ref, sim_ref):
    e1 = e1_ref[0]
    e2 = e2_ref[0]
    ss1 = jnp.sum(e1 * e1)
    ss2 = jnp.sum(e2 * e2)
    inv = 1.0 / jnp.sqrt(ss1 * ss2)
    p = jnp.dot(e1, att_ref[...], preferred_element_type=jnp.float32)
    s_raw = lax.dot_general(
        p, e2, (((1,), (1,)), ((), ())), preferred_element_type=jnp.float32
    )
    s = jnp.tanh(s_raw * inv)
    sim_ref[0] = s

    rm = jnp.sum(s, axis=1, keepdims=True) * (1.0 / L)
    re = jnp.exp(rm - jnp.max(rm))
    rw = re / jnp.sum(re)
    na = lax.dot_general(
        rw, e1, (((0,), (0,)), ((), ())), preferred_element_type=jnp.float32
    )

    cm = jnp.sum(s, axis=0, keepdims=True) * (1.0 / L)
    ce = jnp.exp(cm - jnp.max(cm))
    cw = ce / jnp.sum(ce)
    nb = jnp.dot(cw, e2, preferred_element_type=jnp.float32)

    logit_ref[0, 0, 0] = jnp.sum(na * nb * w_ref[...]) * inv + b_ref[0]


def _tc_body_aliased(e1_ref, e2_ref, att_ref, w_ref, b_ref, simin_ref,
                     login_ref, logit_ref, sim_ref):
    del simin_ref, login_ref
    _tc_body(e1_ref, e2_ref, att_ref, w_ref, b_ref, logit_ref, sim_ref)


_COMMON_IN_SPECS = [
    pl.BlockSpec((1, L, D), lambda b: (b, 0, 0)),
    pl.BlockSpec((1, L, D), lambda b: (b + H, 0, 0)),
    pl.BlockSpec((D, D), lambda b: (0, 0)),
    pl.BlockSpec((1, D), lambda b: (0, 0)),
    pl.BlockSpec(memory_space=pltpu.SMEM),
]

_OUT_SHAPES = [
    jax.ShapeDtypeStruct((B, 1, 1), jnp.float32),
    jax.ShapeDtypeStruct((B, L, L), jnp.float32),
]


def _tc_half_a(g, att_mat, w_row, b_pred):
    return pl.pallas_call(
        _tc_body,
        grid=(H,),
        in_specs=_COMMON_IN_SPECS,
        out_specs=[
            pl.BlockSpec((1, 1, 1), lambda b: (b, 0, 0), memory_space=pltpu.SMEM),
            pl.BlockSpec((1, L, L), lambda b: (b, 0, 0)),
        ],
        out_shape=_OUT_SHAPES,
    )(g, g, att_mat, w_row, b_pred)


def _tc_half_b(g, att_mat, w_row, b_pred, sim_in, log_in):
    return pl.pallas_call(
        _tc_body_aliased,
        grid=(H,),
        in_specs=_COMMON_IN_SPECS + [
            pl.BlockSpec(memory_space=pl.ANY),
            pl.BlockSpec(memory_space=pl.ANY),
        ],
        out_specs=[
            pl.BlockSpec((1, 1, 1), lambda b: (b + H, 0, 0), memory_space=pltpu.SMEM),
            pl.BlockSpec((1, L, L), lambda b: (b + H, 0, 0)),
        ],
        out_shape=_OUT_SHAPES,
        input_output_aliases={5: 1, 6: 0},
    )(g, g, att_mat, w_row, b_pred, sim_in, log_in)


def kernel(t1s, t2s, t1_contexts, t2_contexts, table, att_mat, w_pred, b_pred):
    t1c3 = t1_contexts.astype(jnp.int32).reshape(B, QPI, CHUNK)
    t2c3 = t2_contexts.astype(jnp.int32).reshape(B, QPI, CHUNK)
    ga = _sc_gather_half(table, t1c3, t2c3, 0).reshape(2 * H, L, D)
    gb = _sc_gather_half(table, t1c3, t2c3, H).reshape(2 * H, L, D)
    w_row = w_pred.reshape(1, D)
    la, sim_a = _tc_half_a(ga, att_mat, w_row, b_pred)
    lb, sim = _tc_half_b(gb, att_mat, w_row, b_pred, sim_a, la)
    return lb.reshape(-1), sim


# speedup vs baseline: 1.0532x; 1.0532x over previous
import functools

import jax
import jax.numpy as jnp
from jax import lax
from jax.experimental import pallas as pl
from jax.experimental.pallas import tpu as pltpu
from jax.experimental.pallas import tpu_sc as plsc

B, L, D = 16, 512, 128
H = B // 2
CHUNK = 128
QPI = L // CHUNK


def _sc_gather_half(table, t1r, t2r, b0):
    info = plsc.get_sparse_core_info()
    nc, ns = info.num_cores, info.num_subcores
    mesh = plsc.VectorSubcoreMesh(core_axis_name="c", subcore_axis_name="s")

    @functools.partial(
        pl.kernel,
        mesh=mesh,
        out_type=jax.ShapeDtypeStruct((2, H, L, D), jnp.float32),
        scratch_types=[
            pltpu.VMEM((2, CHUNK), jnp.int32),
            pltpu.VMEM((2 * CHUNK, D), jnp.float32),
            pltpu.SemaphoreType.DMA,
        ],
    )
    def k(table_hbm, t1_hbm, t2_hbm, out_hbm, idx_v, rows_v, sem):
        wid = lax.axis_index("s") * nc + lax.axis_index("c")
        side = wid // 16
        j = wid % 16
        item = j // 2
        half = j % 2
        row0 = (b0 + item) * QPI + half * 2

        @pl.when(side == 0)
        def _():
            pltpu.sync_copy(t1_hbm.at[pl.ds(row0, 2)], idx_v)

        @pl.when(side == 1)
        def _():
            pltpu.sync_copy(t2_hbm.at[pl.ds(row0, 2)], idx_v)

        c1 = pltpu.async_copy(
            table_hbm.at[idx_v.at[0]], rows_v.at[pl.ds(0, CHUNK)], sem)
        c2 = pltpu.async_copy(
            table_hbm.at[idx_v.at[1]], rows_v.at[pl.ds(CHUNK, CHUNK)], sem)
        c1.wait()
        c2.wait()
        pltpu.sync_copy(
            rows_v, out_hbm.at[side, item, pl.ds(half * 2 * CHUNK, 2 * CHUNK)])

    return k(table, t1r, t2r)


def _interact(e1, e2, att, w):
    ss1 = jnp.sum(e1 * e1)
    ss2 = jnp.sum(e2 * e2)
    inv = 1.0 / jnp.sqrt(ss1 * ss2)
    p = jnp.dot(e1, att, preferred_element_type=jnp.float32)
    s_raw = lax.dot_general(
        p, e2, (((1,), (1,)), ((), ())), preferred_element_type=jnp.float32
    )
    s = jnp.tanh(s_raw * inv)

    rm = jnp.sum(s, axis=1, keepdims=True) * (1.0 / L)
    re = jnp.exp(rm - jnp.max(rm))
    rw = re / jnp.sum(re)
    na = lax.dot_general(
        rw, e1, (((0,), (0,)), ((), ())), preferred_element_type=jnp.float32
    )

    cm = jnp.sum(s, axis=0, keepdims=True) * (1.0 / L)
    ce = jnp.exp(cm - jnp.max(cm))
    cw = ce / jnp.sum(ce)
    nb = jnp.dot(cw, e2, preferred_element_type=jnp.float32)

    return na * nb * w * inv, s


def _tc_body_a(e1_ref, e2_ref, att_ref, w_ref, q_ref, sim_ref):
    b = pl.program_id(0)
    q, s = _interact(e1_ref[0], e2_ref[0], att_ref[...], w_ref[...])
    sim_ref[0] = s
    q_ref[pl.ds(b, 1), :] = q


def _tc_body_b(e1_ref, e2_ref, att_ref, w_ref, b_ref, qa_ref, simin_ref,
               logit_ref, sim_ref, qacc_ref):
    del simin_ref
    b = pl.program_id(0)
    q, s = _interact(e1_ref[0], e2_ref[0], att_ref[...], w_ref[...])
    sim_ref[0] = s
    qacc_ref[pl.ds(b, 1), :] = q

    @pl.when(b == H - 1)
    def _():
        qall = jnp.concatenate([qa_ref[...], qacc_ref[...]], axis=0)
        logit_ref[...] = jnp.sum(qall, axis=1) + b_ref[0]


def _tc_half_a(g, att_mat, w_row):
    return pl.pallas_call(
        _tc_body_a,
        grid=(H,),
        in_specs=[
            pl.BlockSpec((1, L, D), lambda b: (b, 0, 0)),
            pl.BlockSpec((1, L, D), lambda b: (b + H, 0, 0)),
            pl.BlockSpec((D, D), lambda b: (0, 0)),
            pl.BlockSpec((1, D), lambda b: (0, 0)),
        ],
        out_specs=[
            pl.BlockSpec((H, D), lambda b: (0, 0)),
            pl.BlockSpec((1, L, L), lambda b: (b, 0, 0)),
        ],
        out_shape=[
            jax.ShapeDtypeStruct((H, D), jnp.float32),
            jax.ShapeDtypeStruct((B, L, L), jnp.float32),
        ],
    )(g, g, att_mat, w_row)


def _tc_half_b(g, att_mat, w_row, b_pred, qa, sim_in):
    return pl.pallas_call(
        _tc_body_b,
        grid=(H,),
        in_specs=[
            pl.BlockSpec((1, L, D), lambda b: (b, 0, 0)),
            pl.BlockSpec((1, L, D), lambda b: (b + H, 0, 0)),
            pl.BlockSpec((D, D), lambda b: (0, 0)),
            pl.BlockSpec((1, D), lambda b: (0, 0)),
            pl.BlockSpec(memory_space=pltpu.SMEM),
            pl.BlockSpec((H, D), lambda b: (0, 0)),
            pl.BlockSpec(memory_space=pl.ANY),
        ],
        out_specs=[
            pl.BlockSpec((B,), lambda b: (0,)),
            pl.BlockSpec((1, L, L), lambda b: (b + H, 0, 0)),
        ],
        out_shape=[
            jax.ShapeDtypeStruct((B,), jnp.float32),
            jax.ShapeDtypeStruct((B, L, L), jnp.float32),
        ],
        scratch_shapes=[pltpu.VMEM((H, D), jnp.float32)],
        input_output_aliases={6: 1},
    )(g, g, att_mat, w_row, b_pred, qa, sim_in)


def kernel(t1s, t2s, t1_contexts, t2_contexts, table, att_mat, w_pred, b_pred):
    t1r = t1_contexts.astype(jnp.int32).reshape(B * QPI, CHUNK)
    t2r = t2_contexts.astype(jnp.int32).reshape(B * QPI, CHUNK)
    ga = _sc_gather_half(table, t1r, t2r, 0).reshape(2 * H, L, D)
    gb = _sc_gather_half(table, t1r, t2r, H).reshape(2 * H, L, D)
    w_row = w_pred.reshape(1, D)
    qa, sim_a = _tc_half_a(ga, att_mat, w_row)
    logits, sim = _tc_half_b(gb, att_mat, w_row, b_pred, qa, sim_a)
    return logits, sim
